# baseline (speedup 1.0000x reference)
import functools

import jax
import jax.numpy as jnp
from jax import lax
from jax.experimental import pallas as pl
from jax.experimental.pallas import tpu as pltpu
from jax.experimental.pallas import tpu_sc as plsc

N = 10000
NPAD = 10240
E = 320000
D = 128
NC = 2
NS = 16
NW = NC * NS
CHUNK = 128
CPW = 80
GRP = 16
EPAD = NW * CPW * CHUNK
RPS = NPAD // NS

_MESH = plsc.VectorSubcoreMesh(
    core_axis_name="c", subcore_axis_name="s", num_cores=NC, num_subcores=NS)


HALF = NPAD // 2
HROWS = HALF * 16 // D
NBLK = NPAD // 1024


def _sc_degrees(src_r, dst_r, zeros, rowidx):
  @functools.partial(
      pl.kernel,
      out_type=jax.ShapeDtypeStruct((NC, 4 * HROWS, D), jnp.float32),
      mesh=_MESH,
      scratch_types=[
          pltpu.VMEM((CPW, CHUNK), jnp.int32),
          pltpu.VMEM((HROWS, D), jnp.float32),
          pltpu.VMEM((4 * HROWS // CHUNK, CHUNK), jnp.int32),
          pltpu.VMEM_SHARED((4 * HROWS, D), jnp.float32),
      ],
      compiler_params=pltpu.CompilerParams(needs_layout_passes=False),
  )
  def k(src_hbm, dst_hbm, zeros_hbm, rowidx_hbm, out_hbm,
        idx_v, hist, rid_v, shist):
    cid = lax.axis_index("c")
    sid = lax.axis_index("s")
    wid = sid * NC + cid
    zrows = 4 * HROWS // NS

    pltpu.sync_copy(zeros_hbm.at[pl.ds(sid * zrows, zrows)],
                    shist.at[pl.ds(sid * zrows, zrows)])
    pltpu.sync_copy(rowidx_hbm, rid_v)
    plsc.subcore_barrier()

    lanes = lax.iota(jnp.int32, 16)
    ones16 = jnp.ones((16,), jnp.float32)

    for which, idx_hbm in ((0, src_hbm), (1, dst_hbm)):
      pltpu.sync_copy(idx_hbm.at[wid], idx_v)
      for half in (0, 1):
        pltpu.sync_copy(zeros_hbm.at[pl.ds(0, HROWS)], hist)

        @pl.loop(0, CPW * CHUNK // 16)
        def _(t, half=half):
          j = t >> 3
          l = (t & 7) * 16
          idx = idx_v[j, pl.ds(l, 16)]
          rel = idx - half * HALF
          m = (rel >= 0) & (rel < HALF)
          flat = jnp.where(m, rel * 16 + lanes, 0)
          plsc.addupdate_scatter(hist, [flat >> 7, flat & 127], ones16,
                                 mask=m)

        base = (which * 2 + half) * HROWS // CHUNK
        for b in range(HROWS // CHUNK):
          pltpu.sync_copy(hist.at[pl.ds(b * CHUNK, CHUNK)],
                          shist.at[rid_v.at[base + b]], add=True)

    plsc.subcore_barrier()

    pltpu.sync_copy(shist.at[pl.ds(sid * zrows, zrows)],
                    out_hbm.at[cid].at[pl.ds(sid * zrows, zrows)])

  return k(src_r, dst_r, zeros, rowidx)


def _sc_aggregate(table, src_r, dst_r, zeros):
  @functools.partial(
      pl.kernel,
      out_type=jax.ShapeDtypeStruct((NC, NPAD, D), jnp.float32),
      mesh=_MESH,
      scratch_types=[
          pltpu.VMEM((GRP, CHUNK), jnp.int32),
          pltpu.VMEM((GRP, CHUNK), jnp.int32),
          pltpu.VMEM((2, CHUNK, D), jnp.float32),
          pltpu.VMEM_SHARED((NPAD, D), jnp.float32),
          pltpu.SemaphoreType.DMA,
          pltpu.SemaphoreType.DMA,
      ],
  )
  def k(tab_hbm, src_hbm, dst_hbm, zeros_hbm, out_hbm,
        sidx, didx, rows, acc, sem0, sem1):
    cid = lax.axis_index("c")
    sid = lax.axis_index("s")
    wid = sid * NC + cid

    pltpu.sync_copy(zeros_hbm.at[pl.ds(sid * RPS, RPS)],
                    acc.at[pl.ds(sid * RPS, RPS)])
    plsc.subcore_barrier()

    SUB = 4
    SLEN = CHUNK // SUB

    def gather(idx_row, buf, sem):
      for s in range(SUB):
        pltpu.async_copy(tab_hbm.at[idx_row.at[pl.ds(s * SLEN, SLEN)]],
                         buf.at[pl.ds(s * SLEN, SLEN)], sem)

    def gather_wait(idx_row, buf, sem):
      for s in range(SUB):
        pltpu.make_async_copy(tab_hbm.at[idx_row.at[pl.ds(s * SLEN, SLEN)]],
                              buf.at[pl.ds(s * SLEN, SLEN)], sem).wait()

    @pl.loop(0, CPW // GRP)
    def _(grp):
      pltpu.sync_copy(src_hbm.at[wid].at[pl.ds(grp * GRP, GRP)], sidx)
      pltpu.sync_copy(dst_hbm.at[wid].at[pl.ds(grp * GRP, GRP)], didx)

      gather(sidx.at[0], rows.at[0], sem0)

      @pl.loop(0, GRP - 2, step=2)
      def _(j):
        gather(sidx.at[j + 1], rows.at[1], sem1)
        gather_wait(sidx.at[j], rows.at[0], sem0)
        pltpu.sync_copy(rows.at[0], acc.at[didx.at[j]], add=True)
        gather(sidx.at[j + 2], rows.at[0], sem0)
        gather_wait(sidx.at[j + 1], rows.at[1], sem1)
        pltpu.sync_copy(rows.at[1], acc.at[didx.at[j + 1]], add=True)

      gather(sidx.at[GRP - 1], rows.at[1], sem1)
      gather_wait(sidx.at[GRP - 2], rows.at[0], sem0)
      pltpu.sync_copy(rows.at[0], acc.at[didx.at[GRP - 2]], add=True)
      gather_wait(sidx.at[GRP - 1], rows.at[1], sem1)
      pltpu.sync_copy(rows.at[1], acc.at[didx.at[GRP - 1]], add=True)

    plsc.subcore_barrier()

    pltpu.sync_copy(acc.at[pl.ds(sid * RPS, RPS)],
                    out_hbm.at[cid].at[pl.ds(sid * RPS, RPS)])

  return k(table, src_r, dst_r, zeros)


_BLK = 1024

_DH_SPEC = pl.BlockSpec((NC, 2, 1, CHUNK, D), lambda i: (0, 0, i, 0, 0))


def _norm(dh_ref, which):
  v = dh_ref[0, which, 0] + dh_ref[1, which, 0]
  n_idx = lax.broadcasted_iota(jnp.int32, (_BLK, D), 0)
  c_idx = lax.broadcasted_iota(jnp.int32, (_BLK, D), 1)
  sel = (n_idx // 8 == c_idx).astype(jnp.float32)
  rep = jnp.dot(sel, v, preferred_element_type=jnp.float32)
  grp = (c_idx // 16 == n_idx % 8).astype(jnp.float32)
  deg = jnp.sum(rep * grp, axis=1, keepdims=True)
  return lax.rsqrt(jnp.maximum(deg, 1.0))


def _tc_mm_scale(x, w, dh):
  def body(x_ref, w_ref, dh_ref, o_ref):
    o_ref[...] = jnp.dot(x_ref[...], w_ref[...],
                         preferred_element_type=jnp.float32) * _norm(dh_ref, 0)

  return pl.pallas_call(
      body,
      grid=(NPAD // _BLK,),
      in_specs=[
          pl.BlockSpec((_BLK, D), lambda i: (i, 0)),
          pl.BlockSpec((D, D), lambda i: (0, 0)),
          _DH_SPEC,
      ],
      out_specs=pl.BlockSpec((_BLK, D), lambda i: (i, 0)),
      out_shape=jax.ShapeDtypeStruct((NPAD, D), jnp.float32),
  )(x, w, dh)


def _tc_mid(p, dh, w2):
  def body(p_ref, dh_ref, w_ref, o_ref):
    h = jnp.maximum((p_ref[0] + p_ref[1]) * _norm(dh_ref, 1), 0.0)
    o_ref[...] = jnp.dot(h, w_ref[...],
                         preferred_element_type=jnp.float32) * _norm(dh_ref, 0)

  return pl.pallas_call(
      body,
      grid=(NPAD // _BLK,),
      in_specs=[
          pl.BlockSpec((NC, _BLK, D), lambda i: (0, i, 0)),
          _DH_SPEC,
          pl.BlockSpec((D, D), lambda i: (0, 0)),
      ],
      out_specs=pl.BlockSpec((_BLK, D), lambda i: (i, 0)),
      out_shape=jax.ShapeDtypeStruct((NPAD, D), jnp.float32),
  )(p, dh, w2)


def _tc_out(q, dh):
  def body(q_ref, dh_ref, o_ref):
    o_ref[...] = (q_ref[0] + q_ref[1]) * _norm(dh_ref, 1)

  return pl.pallas_call(
      body,
      grid=(NPAD // _BLK,),
      in_specs=[
          pl.BlockSpec((NC, _BLK, D), lambda i: (0, i, 0)),
          _DH_SPEC,
      ],
      out_specs=pl.BlockSpec((_BLK, D), lambda i: (i, 0)),
      out_shape=jax.ShapeDtypeStruct((NPAD, D), jnp.float32),
  )(q, dh)


def kernel(g, features, W1, W2):
  src = g[0].astype(jnp.int32)
  dst = g[1].astype(jnp.int32)
  pad = EPAD - E
  fill = jnp.full((pad,), N, jnp.int32)
  src_r = jnp.concatenate([src, fill]).reshape(NW, CPW, CHUNK)
  dst_r = jnp.concatenate([dst, fill]).reshape(NW, CPW, CHUNK)
  xp = jnp.pad(features, ((0, NPAD - N), (0, 0)))
  zeros = jnp.zeros((NPAD, D), jnp.float32)
  rowidx = jnp.arange(4 * HROWS, dtype=jnp.int32).reshape(-1, CHUNK)

  degs = _sc_degrees(src_r, dst_r, zeros, rowidx)
  dh = degs.reshape(NC, 2, NBLK, CHUNK, D)
  t1 = _tc_mm_scale(xp, W1, dh)
  p = _sc_aggregate(t1, src_r, dst_r, zeros)
  t2 = _tc_mid(p, dh, W2)
  q = _sc_aggregate(t2, src_r, dst_r, zeros)
  out = _tc_out(q, dh)
  return out[:N]
